# baseline (speedup 1.0000x reference)
import functools

import jax
import jax.numpy as jnp
from jax import lax
from jax.experimental import pallas as pl
from jax.experimental.pallas import tpu as pltpu
from jax.experimental.pallas import tpu_sc as plsc

_M, _K, _N = 16384, 256, 8192
_TM, _TN = 1024, 512
_GM, _GN = _M // _TM, _N // _TN


def _dist_argmin_body(x_ref, w_ref, x2_ref, w2_ref, idx_ref, bv_ref, bi_ref):
    n = pl.program_id(1)
    xb = x_ref[...]
    wb = w_ref[...]
    acc = jnp.dot(xb, wb, preferred_element_type=jnp.float32)
    d = (x2_ref[...] - 2.0 * acc) + w2_ref[...]
    lmin = jnp.min(d, axis=1, keepdims=True)
    ids = lax.broadcasted_iota(jnp.int32, (_TM, _TN), 1)
    larg = jnp.min(jnp.where(d == lmin, ids, _N), axis=1, keepdims=True) + n * _TN

    @pl.when(n == 0)
    def _():
        bv_ref[...] = lmin
        bi_ref[...] = larg

    @pl.when(n > 0)
    def _():
        bv = bv_ref[...]
        bi = bi_ref[...]
        better = lmin < bv
        bv_ref[...] = jnp.where(better, lmin, bv)
        bi_ref[...] = jnp.where(better, larg, bi)

    @pl.when(n == _GN - 1)
    def _():
        idx_ref[...] = bi_ref[...]


_dist_argmin = pl.pallas_call(
    _dist_argmin_body,
    grid=(_GM, _GN),
    in_specs=[
        pl.BlockSpec((_TM, _K), lambda m, n: (m, 0)),
        pl.BlockSpec((_K, _TN), lambda m, n: (0, n)),
        pl.BlockSpec((_TM, 1), lambda m, n: (m, 0)),
        pl.BlockSpec((1, _TN), lambda m, n: (0, n)),
    ],
    out_specs=pl.BlockSpec((_TM, 1), lambda m, n: (m, 0)),
    out_shape=jax.ShapeDtypeStruct((_M, 1), jnp.int32),
    scratch_shapes=[
        pltpu.VMEM((_TM, 1), jnp.float32),
        pltpu.VMEM((_TM, 1), jnp.int32),
    ],
    compiler_params=pltpu.CompilerParams(
        dimension_semantics=("arbitrary", "arbitrary")
    ),
)


def _transpose_body(w_ref, wt_ref):
    wt_ref[...] = w_ref[...].T


_TT = 1024
_transpose = pl.pallas_call(
    _transpose_body,
    grid=(_N // _TT,),
    in_specs=[pl.BlockSpec((_K, _TT), lambda n: (0, n))],
    out_specs=pl.BlockSpec((_TT, _K), lambda n: (n, 0)),
    out_shape=jax.ShapeDtypeStruct((_N, _K), jnp.float32),
)


_NW = 32
_BPW = _M // _NW
_CH = 128
_NCH = _BPW // _CH


@functools.cache
def _make_gather_sc():
    mesh = plsc.VectorSubcoreMesh(core_axis_name="c", subcore_axis_name="s")

    @functools.partial(
        pl.kernel,
        mesh=mesh,
        out_type=jax.ShapeDtypeStruct((_M, _K), jnp.float32),
        scratch_types=[
            pltpu.VMEM((_NCH, _CH), jnp.int32),
            pltpu.VMEM((_CH, _K), jnp.float32),
            pltpu.SemaphoreType.DMA,
        ],
    )
    def _gather(wt_hbm, idx_hbm, out_hbm, idx_v, rows_v, sem):
        wid = lax.axis_index("s") * 2 + lax.axis_index("c")
        pltpu.sync_copy(idx_hbm.at[pl.ds(wid * _NCH, _NCH)], idx_v)
        for j in range(_NCH):
            pltpu.async_copy(wt_hbm.at[idx_v.at[j]], rows_v, sem).wait()
            pltpu.sync_copy(rows_v, out_hbm.at[pl.ds(wid * _BPW + j * _CH, _CH)])

    return _gather


@jax.jit
def kernel(x, w):
    xf = x.reshape(_M, _K)
    x2 = jnp.sum(xf ** 2, axis=1, keepdims=True)
    w2 = jnp.sum(w ** 2, axis=0, keepdims=True)
    idx = _dist_argmin(xf, w, x2, w2)
    wt = _transpose(w)
    q = _make_gather_sc()(wt, idx.reshape(_M // _CH, _CH))
    return q.reshape(x.shape)
